# baseline (speedup 1.0000x reference)
import jax
import jax.numpy as jnp
from jax import lax
from jax.experimental import pallas as pl
from jax.experimental.pallas import tpu as pltpu
from jax.experimental.pallas import tpu_sc as plsc

B, S, D = 4, 2048, 768
NC, NS = 2, 16
NW = NC * NS
P = S // NW
PC = 8
CH = B * PC
NCHUNK = P // PC
NBUF = 3
LOOKAHEAD = 2
LANES = 16
GROUPS = D // LANES


def _emb_body(xt_hbm, tok_hbm, pos_hbm, out_hbm,
              idx_t, pos_v, buf0, buf1, buf2, gsem, ssem):
    wid = lax.axis_index("s") * NC + lax.axis_index("c")
    p0 = wid * P

    bufs = (buf0, buf1, buf2)

    pltpu.sync_copy(xt_hbm.at[wid], idx_t)

    def start_gather(k):
        return pltpu.async_copy(
            tok_hbm.at[idx_t.at[k]], bufs[k % NBUF], gsem.at[k % NBUF])

    def start_stores(k):
        return [pltpu.async_copy(
                    bufs[k % NBUF].at[pl.ds(b * PC, PC)],
                    out_hbm.at[pl.ds(b * S + p0 + k * PC, PC)],
                    ssem.at[k % NBUF])
                for b in range(B)]

    gathers = [None] * NCHUNK
    stores = [None] * NCHUNK
    gathers[0] = start_gather(0)
    gathers[1] = start_gather(1)
    pltpu.sync_copy(pos_hbm.at[pl.ds(p0, P)], pos_v)

    for k in range(NCHUNK):
        if k + LOOKAHEAD < NCHUNK:
            if k - 1 >= 0:
                for d in stores[k - 1]:
                    d.wait()
            gathers[k + LOOKAHEAD] = start_gather(k + LOOKAHEAD)
        gathers[k].wait()

        buf = bufs[k % NBUF]

        @plsc.parallel_loop(0, PC)
        def add_row(i):
            for j in range(GROUPS):
                sl = pl.ds(j * LANES, LANES)
                pv = pos_v[k * PC + i, sl]
                for b in range(B):
                    buf[b * PC + i, sl] = buf[b * PC + i, sl] + pv

        stores[k] = start_stores(k)
    for k in range(NCHUNK - NBUF, NCHUNK):
        for d in stores[k]:
            d.wait()


@jax.jit
def _emb_call(xt, token_table, pos_table):
    mesh = plsc.VectorSubcoreMesh(core_axis_name="c", subcore_axis_name="s")
    return pl.kernel(
        _emb_body,
        mesh=mesh,
        out_type=jax.ShapeDtypeStruct((B * S, D), jnp.float32),
        scratch_types=[
            pltpu.VMEM((NCHUNK, CH), jnp.int32),
            pltpu.VMEM((P, D), jnp.float32),
            pltpu.VMEM((CH, D), jnp.float32),
            pltpu.VMEM((CH, D), jnp.float32),
            pltpu.VMEM((CH, D), jnp.float32),
            pltpu.SemaphoreType.DMA((NBUF,)),
            pltpu.SemaphoreType.DMA((NBUF,)),
        ],
    )(xt, token_table, pos_table)


def kernel(x, token_table, pos_table):
    xt = (x.astype(jnp.int32)
           .reshape(B, NW, NCHUNK, PC)
           .transpose(1, 2, 0, 3)
           .reshape(NW, NCHUNK, CH))
    out = _emb_call(xt, token_table, pos_table)
    return out.reshape(B, S, D)


# speedup vs baseline: 1.0479x; 1.0479x over previous
import jax
import jax.numpy as jnp
from jax import lax
from jax.experimental import pallas as pl
from jax.experimental.pallas import tpu as pltpu
from jax.experimental.pallas import tpu_sc as plsc

B, S, D = 4, 2048, 768
NC, NS = 2, 16
NW = NC * NS
P = S // NW
PC = 8
CH = B * PC
NCHUNK = P // PC
NBUF = 3
LOOKAHEAD = 2
LANES = 16
GROUPS = D // LANES


def _emb_body(xt_hbm, tok_hbm, pos_hbm, out_hbm,
              idx_t, pos_v, buf0, buf1, buf2, gsem, ssem):
    wid = lax.axis_index("s") * NC + lax.axis_index("c")
    p0 = wid * P

    bufs = (buf0, buf1, buf2)

    pltpu.sync_copy(xt_hbm.at[wid], idx_t)

    def start_gather(k):
        return pltpu.async_copy(
            tok_hbm.at[idx_t.at[k]], bufs[k % NBUF], gsem.at[k % NBUF])

    def start_stores(k):
        return [pltpu.async_copy(
                    bufs[k % NBUF].at[pl.ds(b * PC, PC)],
                    out_hbm.at[pl.ds(b * S + p0 + k * PC, PC)],
                    ssem.at[k % NBUF])
                for b in range(B)]

    gathers = [None] * NCHUNK
    stores = [None] * NCHUNK
    gathers[0] = start_gather(0)
    gathers[1] = start_gather(1)
    pltpu.sync_copy(pos_hbm.at[pl.ds(p0, P)], pos_v)

    for k in range(NCHUNK):
        gathers[k].wait()

        buf = bufs[k % NBUF]

        @plsc.parallel_loop(0, PC)
        def add_row(i):
            for j in range(GROUPS):
                sl = pl.ds(j * LANES, LANES)
                pv = pos_v[k * PC + i, sl]
                for b in range(B):
                    plsc.addupdate(buf.at[b * PC + i, sl], pv)

        stores[k] = start_stores(k)
        if k + LOOKAHEAD < NCHUNK:
            if k - 1 >= 0:
                for d in stores[k - 1]:
                    d.wait()
            gathers[k + LOOKAHEAD] = start_gather(k + LOOKAHEAD)
    for k in range(NCHUNK - NBUF, NCHUNK):
        for d in stores[k]:
            d.wait()


@jax.jit
def _emb_call(xt, token_table, pos_table):
    mesh = plsc.VectorSubcoreMesh(core_axis_name="c", subcore_axis_name="s")
    return pl.kernel(
        _emb_body,
        mesh=mesh,
        out_type=jax.ShapeDtypeStruct((B * S, D), jnp.float32),
        scratch_types=[
            pltpu.VMEM((NCHUNK, CH), jnp.int32),
            pltpu.VMEM((P, D), jnp.float32),
            pltpu.VMEM((CH, D), jnp.float32),
            pltpu.VMEM((CH, D), jnp.float32),
            pltpu.VMEM((CH, D), jnp.float32),
            pltpu.SemaphoreType.DMA((NBUF,)),
            pltpu.SemaphoreType.DMA((NBUF,)),
        ],
    )(xt, token_table, pos_table)


def kernel(x, token_table, pos_table):
    xt = (x.astype(jnp.int32)
           .reshape(B, NW, NCHUNK, PC)
           .transpose(1, 2, 0, 3)
           .reshape(NW, NCHUNK, CH))
    out = _emb_call(xt, token_table, pos_table)
    return out.reshape(B, S, D)


# speedup vs baseline: 1.0675x; 1.0187x over previous
import jax
import jax.numpy as jnp
from jax import lax
from jax.experimental import pallas as pl
from jax.experimental.pallas import tpu as pltpu
from jax.experimental.pallas import tpu_sc as plsc

B, S, D = 4, 2048, 768
NC, NS = 2, 16
NW = NC * NS
P = S // NW
PC = 8
CH = B * PC
NCHUNK = P // PC
NBUF = 3
LOOKAHEAD = 2
LANES = 16
GROUPS = D // LANES


def _emb_body(xt_hbm, tok_hbm, pos_hbm, out_hbm,
              idx_t, pos_v, buf0, buf1, buf2, gsem, ssem):
    wid = lax.axis_index("s") * NC + lax.axis_index("c")
    p0 = wid * P

    bufs = (buf0, buf1, buf2)

    pltpu.sync_copy(xt_hbm.at[wid], idx_t)

    def start_gather(k):
        return pltpu.async_copy(
            tok_hbm.at[idx_t.at[k]], bufs[k % NBUF], gsem.at[k % NBUF])

    def start_stores(k):
        return [pltpu.async_copy(
                    bufs[k % NBUF].at[pl.ds(b * PC, PC)],
                    out_hbm.at[pl.ds(b * S + p0 + k * PC, PC)],
                    ssem.at[k % NBUF])
                for b in range(B)]

    gathers = [None] * NCHUNK
    stores = [None] * NCHUNK
    gathers[0] = start_gather(0)
    gathers[1] = start_gather(1)
    pltpu.sync_copy(pos_hbm.at[pl.ds(p0, P)], pos_v)

    for k in range(NCHUNK):
        if k + LOOKAHEAD < NCHUNK:
            if k - 1 >= 0:
                for d in stores[k - 1]:
                    d.wait()
            gathers[k + LOOKAHEAD] = start_gather(k + LOOKAHEAD)
        gathers[k].wait()

        buf = bufs[k % NBUF]

        @plsc.parallel_loop(0, PC)
        def add_row(i):
            for j in range(GROUPS):
                sl = pl.ds(j * LANES, LANES)
                pv = pos_v[k * PC + i, sl]
                for b in range(B):
                    plsc.addupdate(buf.at[b * PC + i, sl], pv)

        stores[k] = start_stores(k)
    for k in range(NCHUNK - NBUF, NCHUNK):
        for d in stores[k]:
            d.wait()


@jax.jit
def _emb_call(xt, token_table, pos_table):
    mesh = plsc.VectorSubcoreMesh(core_axis_name="c", subcore_axis_name="s")
    return pl.kernel(
        _emb_body,
        mesh=mesh,
        out_type=jax.ShapeDtypeStruct((B * S, D), jnp.float32),
        scratch_types=[
            pltpu.VMEM((NCHUNK, CH), jnp.int32),
            pltpu.VMEM((P, D), jnp.float32),
            pltpu.VMEM((CH, D), jnp.float32),
            pltpu.VMEM((CH, D), jnp.float32),
            pltpu.VMEM((CH, D), jnp.float32),
            pltpu.SemaphoreType.DMA((NBUF,)),
            pltpu.SemaphoreType.DMA((NBUF,)),
        ],
    )(xt, token_table, pos_table)


def kernel(x, token_table, pos_table):
    xt = (x.astype(jnp.int32)
           .reshape(B, NW, NCHUNK, PC)
           .transpose(1, 2, 0, 3)
           .reshape(NW, NCHUNK, CH))
    out = _emb_call(xt, token_table, pos_table)
    return out.reshape(B, S, D)
